# baseline (speedup 1.0000x reference)
import functools

import jax
import jax.numpy as jnp
from jax import lax
from jax.experimental import pallas as pl
from jax.experimental.pallas import tpu as pltpu
from jax.experimental.pallas import tpu_sc as plsc

_K = 1024
_M = 1024
_B = 8
_C = 384
_N = 1024
_CPW = 16
_AW = _C // _CPW
_L = 16


def _argmin_body(flat_ref, cb_ref, xsq_ref, cbsq_ref, idx_ref):
    ft = flat_ref[...]
    cb = cb_ref[...]
    cross = lax.dot_general(ft, cb, (((1,), (1,)), ((), ())),
                            preferred_element_type=jnp.float32)
    t = (xsq_ref[...] + cbsq_ref[...]) - 2.0 * cross
    dist = jnp.sqrt(jnp.clip(t, 0.0, None))
    minv = jnp.min(dist, axis=1, keepdims=True)
    kiota = lax.broadcasted_iota(jnp.int32, (_M, _K), 1)
    masked = jnp.where(dist == minv, kiota, _K)
    idx_ref[...] = jnp.min(masked, axis=1, keepdims=True)


def _sc_gather(cbt_flat, idx_flat):
    mesh = plsc.VectorSubcoreMesh(core_axis_name="c", subcore_axis_name="s")

    @functools.partial(
        pl.kernel,
        mesh=mesh,
        out_type=jax.ShapeDtypeStruct((_B * _C * _N,), jnp.float32),
        compiler_params=pltpu.CompilerParams(needs_layout_passes=False),
        scratch_types=[
            pltpu.VMEM((_CPW * _K,), jnp.float32),
            pltpu.VMEM((_B * _N,), jnp.int32),
            pltpu.VMEM((_CPW * _N,), jnp.float32),
        ],
    )
    def k(cbt_hbm, idx_hbm, q_hbm, tab_v, idx_v, out_v):
        wid = lax.axis_index("s") * 2 + lax.axis_index("c")

        @pl.when(wid < _AW)
        def _():
            cs = wid * _CPW
            pltpu.sync_copy(cbt_hbm.at[pl.ds(cs * _K, _CPW * _K)], tab_v)
            pltpu.sync_copy(idx_hbm, idx_v)

            def batch_body(b, carry):
                def row_body(r, c2):
                    def vec_body(i, c3):
                        idxs = idx_v[pl.ds(b * _N + i * _L, _L)]
                        out_v[pl.ds(r * _N + i * _L, _L)] = plsc.load_gather(
                            tab_v, [r * _K + idxs])
                        return c3

                    return lax.fori_loop(0, _N // _L, vec_body, c2)

                lax.fori_loop(0, _CPW, row_body, 0)
                pltpu.sync_copy(
                    out_v,
                    q_hbm.at[pl.ds(b * _C * _N + cs * _N, _CPW * _N)])
                return carry

            lax.fori_loop(0, _B, batch_body, 0)

    return k(cbt_flat, idx_flat)


def kernel(x, codebook):
    b, c, h, w = x.shape
    n = h * w
    flat = jnp.transpose(x, (0, 2, 3, 1)).reshape(b, n, c).astype(jnp.float32)
    x_sq = jnp.sum(flat * flat, axis=-1, keepdims=True)
    cb_sq = jnp.sum(codebook * codebook, axis=-1)
    rows = b * n
    flat2 = flat.reshape(rows, c)
    xsq2 = x_sq.reshape(rows, 1)
    cbsq2 = cb_sq.reshape(1, _K)
    grid = rows // _M
    idx = pl.pallas_call(
        _argmin_body,
        grid=(grid,),
        in_specs=[
            pl.BlockSpec((_M, c), lambda i: (i, 0)),
            pl.BlockSpec((_K, c), lambda i: (0, 0)),
            pl.BlockSpec((_M, 1), lambda i: (i, 0)),
            pl.BlockSpec((1, _K), lambda i: (0, 0)),
        ],
        out_specs=pl.BlockSpec((_M, 1), lambda i: (i, 0)),
        out_shape=jax.ShapeDtypeStruct((rows, 1), jnp.int32),
    )(flat2, codebook, xsq2, cbsq2)
    idx2 = idx.reshape(b, n)
    cbt = jnp.transpose(codebook)
    qf = _sc_gather(cbt.reshape(-1), idx2.reshape(-1))
    quantized = qf.reshape(b, c, h, w)
    embed_index = idx2.reshape(b, h, w)
    loss = jnp.array([0.0], dtype=jnp.float32)
    return (quantized, embed_index, loss)


# speedup vs baseline: 1.5876x; 1.5876x over previous
import functools

import jax
import jax.numpy as jnp
from jax import lax
from jax.experimental import pallas as pl
from jax.experimental.pallas import tpu as pltpu
from jax.experimental.pallas import tpu_sc as plsc

_K = 1024
_M = 1024
_B = 8
_C = 384
_N = 1024
_CPW = 16
_AW = _C // _CPW
_L = 16


def _argmin_body(flat_ref, cb_ref, xsq_ref, cbsq_ref, idx_ref):
    ft = flat_ref[...]
    cb = cb_ref[...]
    cross = lax.dot_general(ft, cb, (((1,), (1,)), ((), ())),
                            preferred_element_type=jnp.float32)
    t = (xsq_ref[...] + cbsq_ref[...]) - 2.0 * cross
    dist = jnp.sqrt(jnp.clip(t, 0.0, None))
    minv = jnp.min(dist, axis=1, keepdims=True)
    kiota = lax.broadcasted_iota(jnp.int32, (_M, _K), 1)
    masked = jnp.where(dist == minv, kiota, _K)
    idx_ref[...] = jnp.min(masked, axis=1, keepdims=True)


def _sc_gather(cbt_flat, idx_flat):
    mesh = plsc.VectorSubcoreMesh(core_axis_name="c", subcore_axis_name="s")

    @functools.partial(
        pl.kernel,
        mesh=mesh,
        out_type=jax.ShapeDtypeStruct((_B * _C * _N,), jnp.float32),
        compiler_params=pltpu.CompilerParams(needs_layout_passes=False),
        scratch_types=[
            pltpu.VMEM((_CPW * _K,), jnp.float32),
            pltpu.VMEM((_B * _N,), jnp.int32),
            pltpu.VMEM((2 * _CPW * _N,), jnp.float32),
            pltpu.SemaphoreType.DMA,
            pltpu.SemaphoreType.DMA,
        ],
    )
    def k(cbt_hbm, idx_hbm, q_hbm, tab_v, idx_v, out_v, sem0, sem1):
        wid = lax.axis_index("s") * 2 + lax.axis_index("c")

        @pl.when(wid < _AW)
        def _():
            cs = wid * _CPW
            pltpu.sync_copy(cbt_hbm.at[pl.ds(cs * _K, _CPW * _K)], tab_v)
            pltpu.sync_copy(idx_hbm, idx_v)
            sems = (sem0, sem1)
            descs = [None, None]
            for b in range(_B):
                slot = b % 2
                if descs[slot] is not None:
                    descs[slot].wait()
                ob = slot * _CPW * _N

                @plsc.parallel_loop(0, _N // _L, unroll=4)
                def body(i):
                    idxs = idx_v[pl.ds(b * _N + i * _L, _L)]
                    for r in range(_CPW):
                        out_v[pl.ds(ob + r * _N + i * _L, _L)] = (
                            plsc.load_gather(tab_v, [idxs + r * _K]))

                descs[slot] = pltpu.async_copy(
                    out_v.at[pl.ds(ob, _CPW * _N)],
                    q_hbm.at[pl.ds(b * _C * _N + cs * _N, _CPW * _N)],
                    sems[slot])
            descs[0].wait()
            descs[1].wait()

    return k(cbt_flat, idx_flat)


def kernel(x, codebook):
    b, c, h, w = x.shape
    n = h * w
    flat = jnp.transpose(x, (0, 2, 3, 1)).reshape(b, n, c).astype(jnp.float32)
    x_sq = jnp.sum(flat * flat, axis=-1, keepdims=True)
    cb_sq = jnp.sum(codebook * codebook, axis=-1)
    rows = b * n
    flat2 = flat.reshape(rows, c)
    xsq2 = x_sq.reshape(rows, 1)
    cbsq2 = cb_sq.reshape(1, _K)
    grid = rows // _M
    idx = pl.pallas_call(
        _argmin_body,
        grid=(grid,),
        in_specs=[
            pl.BlockSpec((_M, c), lambda i: (i, 0)),
            pl.BlockSpec((_K, c), lambda i: (0, 0)),
            pl.BlockSpec((_M, 1), lambda i: (i, 0)),
            pl.BlockSpec((1, _K), lambda i: (0, 0)),
        ],
        out_specs=pl.BlockSpec((_M, 1), lambda i: (i, 0)),
        out_shape=jax.ShapeDtypeStruct((rows, 1), jnp.int32),
    )(flat2, codebook, xsq2, cbsq2)
    idx2 = idx.reshape(b, n)
    cbt = jnp.transpose(codebook)
    qf = _sc_gather(cbt.reshape(-1), idx2.reshape(-1))
    quantized = qf.reshape(b, c, h, w)
    embed_index = idx2.reshape(b, h, w)
    loss = jnp.array([0.0], dtype=jnp.float32)
    return (quantized, embed_index, loss)
